# baseline (speedup 1.0000x reference)
import functools

import jax
import jax.numpy as jnp
from jax import lax
from jax.experimental import pallas as pl
from jax.experimental.pallas import tpu as pltpu
from jax.experimental.pallas import tpu_sc as plsc

N = 10000
D = 128
NC = 2
NS = 16
CHUNK = 128
ROWS_PER_TILE = 640
LANES = 16


def _spmm_partials(src2d, dst2d, w2d, h, zeros, n_chunks):
    mesh = plsc.VectorSubcoreMesh(core_axis_name="c", subcore_axis_name="s",
                                  num_cores=NC, num_subcores=NS)
    hp = n_chunks // 2

    @functools.partial(
        pl.kernel,
        out_type=jax.ShapeDtypeStruct((NC, N, D), jnp.float32),
        mesh=mesh,
        scratch_types=[
            pltpu.VMEM((hp, CHUNK), jnp.int32),
            pltpu.VMEM((hp, CHUNK), jnp.int32),
            pltpu.VMEM((hp, CHUNK), jnp.float32),
            pltpu.VMEM((CHUNK, D), jnp.float32),
            pltpu.VMEM((CHUNK, D), jnp.float32),
            pltpu.VMEM_SHARED((N, D), jnp.float32),
            pltpu.SemaphoreType.DMA,
            pltpu.SemaphoreType.DMA,
            pltpu.SemaphoreType.DMA,
            pltpu.SemaphoreType.DMA,
        ],
    )
    def spmm(src_hbm, dst_hbm, w_hbm, h_hbm, z_hbm, out_hbm,
             sidx_all, didx_all, w_all, rows0, rows1, acc_sh,
             gsem0, gsem1, ssem0, ssem1):
        c = lax.axis_index("c")
        s = lax.axis_index("s")
        wid = c * NS + s
        r0 = pl.multiple_of(
            jnp.minimum(s * ROWS_PER_TILE, N - ROWS_PER_TILE), 8)
        pltpu.sync_copy(z_hbm.at[pl.ds(r0, ROWS_PER_TILE)],
                        acc_sh.at[pl.ds(r0, ROWS_PER_TILE)])
        plsc.subcore_barrier()

        def gather(g, rows, gsem):
            pltpu.async_copy(h_hbm.at[sidx_all.at[g]], rows, gsem)

        def gather_wait(g, rows, gsem):
            pltpu.make_async_copy(h_hbm.at[sidx_all.at[g]], rows, gsem).wait()

        def scatter(g, rows, ssem):
            pltpu.async_copy(rows, acc_sh.at[didx_all.at[g]], ssem, add=True)

        def scatter_wait(g, rows, ssem):
            pltpu.make_async_copy(rows, acc_sh.at[didx_all.at[g]],
                                  ssem).wait()

        def scale(g, rows):
            def group_body(g16, c2):
                w16 = w_all[g, pl.ds(g16 * LANES, LANES)]
                for k in range(LANES):
                    wi = w16[k]
                    i = g16 * LANES + k
                    for j in range(D // LANES):
                        sl = pl.ds(j * LANES, LANES)
                        rows[i, sl] = rows[i, sl] * wi
                return c2

            lax.fori_loop(0, CHUNK // LANES, group_body, 0)

        rows = (rows0, rows1)
        gsems = (gsem0, gsem1)
        ssems = (ssem0, ssem1)

        def run_phase(p):
            cb = pl.multiple_of(wid * n_chunks + p * hp, 8)
            pltpu.sync_copy(src_hbm.at[pl.ds(cb, hp)], sidx_all)
            pltpu.sync_copy(dst_hbm.at[pl.ds(cb, hp)], didx_all)
            pltpu.sync_copy(w_hbm.at[pl.ds(cb, hp)], w_all)

            gather(0, rows0, gsem0)

            def pair_body(t, carry):
                for b in range(2):
                    g = t * 2 + b
                    ob = 1 - b

                    def retire(g=g, ob=ob):
                        scatter_wait(g - 1, rows[ob], ssems[ob])

                    if b == 0:
                        pl.when(t > 0)(retire)
                    else:
                        retire()

                    def prefetch(g=g, ob=ob):
                        gather(g + 1, rows[ob], gsems[ob])

                    if b == 1:
                        pl.when(g + 1 < hp)(prefetch)
                    else:
                        prefetch()

                    gather_wait(g, rows[b], gsems[b])
                    scale(g, rows[b])
                    scatter(g, rows[b], ssems[b])
                return carry

            lax.fori_loop(0, hp // 2, pair_body, 0)
            scatter_wait(hp - 1, rows1, ssem1)

        for p in range(2):
            run_phase(p)
        plsc.subcore_barrier()
        pltpu.sync_copy(acc_sh.at[pl.ds(r0, ROWS_PER_TILE)],
                        out_hbm.at[c, pl.ds(r0, ROWS_PER_TILE)])

    return spmm(src2d, dst2d, w2d, h, zeros)


ROW_BLK = 1000


def _matmul_kernel(x_ref, w_ref, o_ref):
    o_ref[...] = jnp.dot(x_ref[...], w_ref[...],
                         preferred_element_type=jnp.float32)


def _matmul(x, w):
    return pl.pallas_call(
        _matmul_kernel,
        out_shape=jax.ShapeDtypeStruct((N, D), jnp.float32),
        grid=(N // ROW_BLK,),
        in_specs=[
            pl.BlockSpec((ROW_BLK, D), lambda i: (i, 0)),
            pl.BlockSpec((D, D), lambda i: (0, 0)),
        ],
        out_specs=pl.BlockSpec((ROW_BLK, D), lambda i: (i, 0)),
    )(x, w)


def _fuse_relu_matmul_kernel(p0_ref, p1_ref, b_ref, w_ref, o_ref):
    h = jnp.maximum(p0_ref[...] + p1_ref[...] + b_ref[...], 0.0)
    o_ref[...] = jnp.dot(h, w_ref[...], preferred_element_type=jnp.float32)


def _fuse_relu_matmul(p0, p1, b, w):
    return pl.pallas_call(
        _fuse_relu_matmul_kernel,
        out_shape=jax.ShapeDtypeStruct((N, D), jnp.float32),
        grid=(N // ROW_BLK,),
        in_specs=[
            pl.BlockSpec((ROW_BLK, D), lambda i: (i, 0)),
            pl.BlockSpec((ROW_BLK, D), lambda i: (i, 0)),
            pl.BlockSpec((1, D), lambda i: (0, 0)),
            pl.BlockSpec((D, D), lambda i: (0, 0)),
        ],
        out_specs=pl.BlockSpec((ROW_BLK, D), lambda i: (i, 0)),
    )(p0, p1, b, w)


def _add_bias_kernel(p0_ref, p1_ref, b_ref, o_ref):
    o_ref[...] = p0_ref[...] + p1_ref[...] + b_ref[...]


def _add_bias(p0, p1, b):
    return pl.pallas_call(
        _add_bias_kernel,
        out_shape=jax.ShapeDtypeStruct((N, D), jnp.float32),
        grid=(N // ROW_BLK,),
        in_specs=[
            pl.BlockSpec((ROW_BLK, D), lambda i: (i, 0)),
            pl.BlockSpec((ROW_BLK, D), lambda i: (i, 0)),
            pl.BlockSpec((1, D), lambda i: (0, 0)),
        ],
        out_specs=pl.BlockSpec((ROW_BLK, D), lambda i: (i, 0)),
    )(p0, p1, b)


def kernel(x, edge_index, edge_weight, W1, b1, W2, b2):
    e = edge_index.shape[1]
    n_chunks = -(-(-(-e // (NC * NS * CHUNK))) // 16) * 16
    ep = NC * NS * CHUNK * n_chunks
    pad = ep - e
    src = jnp.pad(edge_index[0], (0, pad)).reshape(-1, CHUNK)
    dst = jnp.pad(edge_index[1], (0, pad)).reshape(-1, CHUNK)
    w = jnp.pad(edge_weight, (0, pad)).reshape(-1, CHUNK)
    zeros = jnp.zeros((N, D), jnp.float32)

    h0 = _matmul(x, W1)
    p = _spmm_partials(src, dst, w, h0, zeros, n_chunks)
    h1 = _fuse_relu_matmul(p[0], p[1], b1.reshape(1, D), W2)
    q = _spmm_partials(src, dst, w, h1, zeros, n_chunks)
    return _add_bias(q[0], q[1], b2.reshape(1, D))


# speedup vs baseline: 1.0700x; 1.0700x over previous
import functools

import jax
import jax.numpy as jnp
from jax import lax
from jax.experimental import pallas as pl
from jax.experimental.pallas import tpu as pltpu
from jax.experimental.pallas import tpu_sc as plsc

N = 10000
D = 128
NC = 2
NS = 16
CHUNK = 64
ROWS_PER_TILE = 640
LANES = 16


def _spmm_partials(src2d, dst2d, w2d, h, zeros, n_chunks):
    mesh = plsc.VectorSubcoreMesh(core_axis_name="c", subcore_axis_name="s",
                                  num_cores=NC, num_subcores=NS)
    hp = n_chunks // 4

    @functools.partial(
        pl.kernel,
        out_type=jax.ShapeDtypeStruct((NC, N, D), jnp.float32),
        mesh=mesh,
        scratch_types=[
            pltpu.VMEM((hp, CHUNK), jnp.int32),
            pltpu.VMEM((hp, CHUNK), jnp.int32),
            pltpu.VMEM((hp, CHUNK), jnp.float32),
            pltpu.VMEM((CHUNK, D), jnp.float32),
            pltpu.VMEM((CHUNK, D), jnp.float32),
            pltpu.VMEM((CHUNK, D), jnp.float32),
            pltpu.VMEM((CHUNK, D), jnp.float32),
            pltpu.VMEM_SHARED((N, D), jnp.float32),
            pltpu.SemaphoreType.DMA,
            pltpu.SemaphoreType.DMA,
            pltpu.SemaphoreType.DMA,
            pltpu.SemaphoreType.DMA,
            pltpu.SemaphoreType.DMA,
            pltpu.SemaphoreType.DMA,
            pltpu.SemaphoreType.DMA,
            pltpu.SemaphoreType.DMA,
        ],
    )
    def spmm(src_hbm, dst_hbm, w_hbm, h_hbm, z_hbm, out_hbm,
             sidx_all, didx_all, w_all, rows0, rows1, rows2, rows3, acc_sh,
             gsem0, gsem1, gsem2, gsem3, ssem0, ssem1, ssem2, ssem3):
        c = lax.axis_index("c")
        s = lax.axis_index("s")
        wid = c * NS + s
        r0 = pl.multiple_of(
            jnp.minimum(s * ROWS_PER_TILE, N - ROWS_PER_TILE), 8)
        pltpu.sync_copy(z_hbm.at[pl.ds(r0, ROWS_PER_TILE)],
                        acc_sh.at[pl.ds(r0, ROWS_PER_TILE)])
        plsc.subcore_barrier()

        def gather(g, rows, gsem):
            pltpu.async_copy(h_hbm.at[sidx_all.at[g, pl.ds(0, 32)]],
                             rows.at[pl.ds(0, 32)], gsem)
            pltpu.async_copy(h_hbm.at[sidx_all.at[g, pl.ds(32, 32)]],
                             rows.at[pl.ds(32, 32)], gsem)

        def gather_wait(g, rows, gsem):
            pltpu.make_async_copy(h_hbm.at[sidx_all.at[g, pl.ds(0, 32)]],
                                  rows.at[pl.ds(0, 32)], gsem).wait()
            pltpu.make_async_copy(h_hbm.at[sidx_all.at[g, pl.ds(32, 32)]],
                                  rows.at[pl.ds(32, 32)], gsem).wait()

        def scatter(g, rows, ssem):
            pltpu.async_copy(rows, acc_sh.at[didx_all.at[g]], ssem, add=True)

        def scatter_wait(g, rows, ssem):
            pltpu.make_async_copy(rows, acc_sh.at[didx_all.at[g]],
                                  ssem).wait()

        def scale(g, rows):
            def group_body(g16, c2):
                w16 = w_all[g, pl.ds(g16 * LANES, LANES)]
                for k in range(LANES):
                    wi = w16[k]
                    i = g16 * LANES + k
                    for j in range(D // LANES):
                        sl = pl.ds(j * LANES, LANES)
                        rows[i, sl] = rows[i, sl] * wi
                return c2

            lax.fori_loop(0, CHUNK // LANES, group_body, 0)

        rows = (rows0, rows1, rows2, rows3)
        gsems = (gsem0, gsem1, gsem2, gsem3)
        ssems = (ssem0, ssem1, ssem2, ssem3)

        def run_phase(p):
            cb = pl.multiple_of(wid * n_chunks + p * hp, 8)
            pltpu.sync_copy(src_hbm.at[pl.ds(cb, hp)], sidx_all)
            pltpu.sync_copy(dst_hbm.at[pl.ds(cb, hp)], didx_all)
            pltpu.sync_copy(w_hbm.at[pl.ds(cb, hp)], w_all)

            for b in range(4):
                gather(b, rows[b], gsems[b])

            def quad_body(t, carry):
                for b in range(4):
                    g = t * 4 + b
                    gather_wait(g, rows[b], gsems[b])
                    scale(g, rows[b])
                    scatter(g, rows[b], ssems[b])
                    br = (b + 2) % 4
                    gr = g - 2

                    def retire(gr=gr, br=br):
                        scatter_wait(gr, rows[br], ssems[br])
                        gather(gr + 4, rows[br], gsems[br])

                    if b < 2:
                        pl.when(t > 0)(retire)
                    else:
                        pl.when(gr + 4 < hp)(retire)

                        def drain_only(gr=gr, br=br):
                            scatter_wait(gr, rows[br], ssems[br])

                        pl.when(gr + 4 >= hp)(drain_only)
                return carry

            lax.fori_loop(0, hp // 4, quad_body, 0)
            scatter_wait(hp - 2, rows2, ssem2)
            scatter_wait(hp - 1, rows3, ssem3)

        for p in range(4):
            run_phase(p)
        plsc.subcore_barrier()
        pltpu.sync_copy(acc_sh.at[pl.ds(r0, ROWS_PER_TILE)],
                        out_hbm.at[c, pl.ds(r0, ROWS_PER_TILE)])

    return spmm(src2d, dst2d, w2d, h, zeros)


ROW_BLK = 1000


def _matmul_kernel(x_ref, w_ref, o_ref):
    o_ref[...] = jnp.dot(x_ref[...], w_ref[...],
                         preferred_element_type=jnp.float32)


def _matmul(x, w):
    return pl.pallas_call(
        _matmul_kernel,
        out_shape=jax.ShapeDtypeStruct((N, D), jnp.float32),
        grid=(N // ROW_BLK,),
        in_specs=[
            pl.BlockSpec((ROW_BLK, D), lambda i: (i, 0)),
            pl.BlockSpec((D, D), lambda i: (0, 0)),
        ],
        out_specs=pl.BlockSpec((ROW_BLK, D), lambda i: (i, 0)),
    )(x, w)


def _fuse_relu_matmul_kernel(p0_ref, p1_ref, b_ref, w_ref, o_ref):
    h = jnp.maximum(p0_ref[...] + p1_ref[...] + b_ref[...], 0.0)
    o_ref[...] = jnp.dot(h, w_ref[...], preferred_element_type=jnp.float32)


def _fuse_relu_matmul(p0, p1, b, w):
    return pl.pallas_call(
        _fuse_relu_matmul_kernel,
        out_shape=jax.ShapeDtypeStruct((N, D), jnp.float32),
        grid=(N // ROW_BLK,),
        in_specs=[
            pl.BlockSpec((ROW_BLK, D), lambda i: (i, 0)),
            pl.BlockSpec((ROW_BLK, D), lambda i: (i, 0)),
            pl.BlockSpec((1, D), lambda i: (0, 0)),
            pl.BlockSpec((D, D), lambda i: (0, 0)),
        ],
        out_specs=pl.BlockSpec((ROW_BLK, D), lambda i: (i, 0)),
    )(p0, p1, b, w)


def _add_bias_kernel(p0_ref, p1_ref, b_ref, o_ref):
    o_ref[...] = p0_ref[...] + p1_ref[...] + b_ref[...]


def _add_bias(p0, p1, b):
    return pl.pallas_call(
        _add_bias_kernel,
        out_shape=jax.ShapeDtypeStruct((N, D), jnp.float32),
        grid=(N // ROW_BLK,),
        in_specs=[
            pl.BlockSpec((ROW_BLK, D), lambda i: (i, 0)),
            pl.BlockSpec((ROW_BLK, D), lambda i: (i, 0)),
            pl.BlockSpec((1, D), lambda i: (0, 0)),
        ],
        out_specs=pl.BlockSpec((ROW_BLK, D), lambda i: (i, 0)),
    )(p0, p1, b)


def kernel(x, edge_index, edge_weight, W1, b1, W2, b2):
    e = edge_index.shape[1]
    n_chunks = -(-(-(-e // (NC * NS * CHUNK))) // 32) * 32
    ep = NC * NS * CHUNK * n_chunks
    pad = ep - e
    src = jnp.pad(edge_index[0], (0, pad)).reshape(-1, CHUNK)
    dst = jnp.pad(edge_index[1], (0, pad)).reshape(-1, CHUNK)
    w = jnp.pad(edge_weight, (0, pad)).reshape(-1, CHUNK)
    zeros = jnp.zeros((N, D), jnp.float32)

    h0 = _matmul(x, W1)
    p = _spmm_partials(src, dst, w, h0, zeros, n_chunks)
    h1 = _fuse_relu_matmul(p[0], p[1], b1.reshape(1, D), W2)
    q = _spmm_partials(src, dst, w, h1, zeros, n_chunks)
    return _add_bias(q[0], q[1], b2.reshape(1, D))
